# Initial kernel scaffold; baseline (speedup 1.0000x reference)
import jax
import jax.numpy as jnp
from jax.experimental import pallas as pl


def kernel(inputs, pos_table):
    raise NotImplementedError("write your pallas kernel here")


# speedup vs baseline: 1.6646x; 1.6646x over previous
import jax
import jax.numpy as jnp
from jax.experimental import pallas as pl


_BS = 256


def _add_kernel(x_ref, t_ref, o_ref):
    o_ref[...] = x_ref[...] + t_ref[...]


def kernel(inputs, pos_table):
    batch, seq, dm = inputs.shape
    nblk = seq // _BS
    return pl.pallas_call(
        _add_kernel,
        grid=(nblk, batch),
        in_specs=[
            pl.BlockSpec((1, _BS, dm), lambda i, b: (b, i, 0)),
            pl.BlockSpec((_BS, dm), lambda i, b: (i, 0)),
        ],
        out_specs=pl.BlockSpec((1, _BS, dm), lambda i, b: (b, i, 0)),
        out_shape=jax.ShapeDtypeStruct(inputs.shape, inputs.dtype),
    )(inputs, pos_table)


# speedup vs baseline: 1.8372x; 1.1037x over previous
import jax
import jax.numpy as jnp
from jax.experimental import pallas as pl


_BS = 512


def _add_kernel(x_ref, t_ref, o_ref):
    o_ref[...] = x_ref[...] + t_ref[...]


def kernel(inputs, pos_table):
    batch, seq, dm = inputs.shape
    nblk = seq // _BS
    return pl.pallas_call(
        _add_kernel,
        grid=(nblk, batch),
        in_specs=[
            pl.BlockSpec((1, _BS, dm), lambda i, b: (b, i, 0)),
            pl.BlockSpec((_BS, dm), lambda i, b: (i, 0)),
        ],
        out_specs=pl.BlockSpec((1, _BS, dm), lambda i, b: (b, i, 0)),
        out_shape=jax.ShapeDtypeStruct(inputs.shape, inputs.dtype),
    )(inputs, pos_table)


# speedup vs baseline: 1.9697x; 1.0721x over previous
import jax
import jax.numpy as jnp
from jax.experimental import pallas as pl


_BS = 1024


def _add_kernel(x_ref, t_ref, o_ref):
    o_ref[...] = x_ref[...] + t_ref[...]


def kernel(inputs, pos_table):
    batch, seq, dm = inputs.shape
    nblk = seq // _BS
    return pl.pallas_call(
        _add_kernel,
        grid=(nblk, batch),
        in_specs=[
            pl.BlockSpec((1, _BS, dm), lambda i, b: (b, i, 0)),
            pl.BlockSpec((_BS, dm), lambda i, b: (i, 0)),
        ],
        out_specs=pl.BlockSpec((1, _BS, dm), lambda i, b: (b, i, 0)),
        out_shape=jax.ShapeDtypeStruct(inputs.shape, inputs.dtype),
    )(inputs, pos_table)
